# baseline (speedup 1.0000x reference)
import math

import jax
import jax.numpy as jnp
from jax.experimental import pallas as pl
from jax.experimental.pallas import tpu as pltpu

V = 32000
SMOOTH_A = 0.1 / (V - 1)
CONF_C = 0.9
K_CONST = (V - 1) * SMOOTH_A * math.log(SMOOTH_A) + CONF_C * math.log(CONF_C)

R_BLK = 2048
V_BLK = 3200
N_ROWS = 4096
NR = N_ROWS // R_BLK
NV = V // V_BLK


def _loss_body(tgt_ref, pred_ref, out_ref, acc_ref, cnt_ref):
    i = pl.program_id(0)
    j = pl.program_id(1)

    @pl.when((i == 0) & (j == 0))
    def _init():
        acc_ref[0] = 0.0
        cnt_ref[0] = 0.0

    tgt = tgt_ref[0, 0, :]
    maskf = (tgt > 0).astype(jnp.float32)

    @pl.when(j == 0)
    def _count():
        cnt_ref[0] += jnp.sum(maskf)

    pred = pred_ref[...]
    tloc = tgt - j * V_BLK
    af = maskf * SMOOTH_A
    cf = maskf * CONF_C
    col0 = jax.lax.broadcasted_iota(jnp.int32, (R_BLK, V_BLK), 1)
    w = jnp.where(col0 == tloc[:, None], cf[:, None], af[:, None])
    acc_ref[0] += jnp.sum(pred * w)

    @pl.when((i == NR - 1) & (j == NV - 1))
    def _fin():
        out_ref[0] = K_CONST - acc_ref[0] / cnt_ref[0]


def kernel(prediction, target):
    pred = prediction.reshape(N_ROWS, V)
    tgt = target.reshape(NR, 1, R_BLK).astype(jnp.int32)
    out = pl.pallas_call(
        _loss_body,
        grid=(NR, NV),
        in_specs=[
            pl.BlockSpec((1, 1, R_BLK), lambda i, j: (i, 0, 0)),
            pl.BlockSpec((R_BLK, V_BLK), lambda i, j: (i, j)),
        ],
        out_specs=pl.BlockSpec(memory_space=pltpu.SMEM),
        out_shape=jax.ShapeDtypeStruct((1,), jnp.float32),
        scratch_shapes=[
            pltpu.SMEM((1,), jnp.float32),
            pltpu.SMEM((1,), jnp.float32),
        ],
    )(tgt, pred)
    return out[0]


# speedup vs baseline: 1.0659x; 1.0659x over previous
import math

import jax
import jax.numpy as jnp
from jax.experimental import pallas as pl
from jax.experimental.pallas import tpu as pltpu

V = 32000
SMOOTH_A = 0.1 / (V - 1)
CONF_C = 0.9
K_CONST = (V - 1) * SMOOTH_A * math.log(SMOOTH_A) + CONF_C * math.log(CONF_C)

R_BLK = 2048
V_BLK = 3200
N_ROWS = 4096
NR = N_ROWS // R_BLK
NV = V // V_BLK


def _loss_body(tgt_ref, pred_ref, out_ref, acc_ref, cnt_ref):
    i = pl.program_id(0)
    j = pl.program_id(1)

    @pl.when((i == 0) & (j == 0))
    def _init():
        acc_ref[0] = 0.0
        cnt_ref[0] = 0.0

    tgt = tgt_ref[0, 0, :]
    maskf = (tgt > 0).astype(jnp.float32)

    @pl.when(j == 0)
    def _count():
        cnt_ref[0] += jnp.sum(maskf)

    pred = pred_ref[...]
    tloc = tgt - j * V_BLK
    col0 = jax.lax.broadcasted_iota(jnp.int32, (R_BLK, V_BLK), 1)
    w = jnp.where(col0 == tloc[:, None], CONF_C, SMOOTH_A)
    row_part = jnp.sum(pred * w, axis=1)
    acc_ref[0] += jnp.sum(row_part * maskf)

    @pl.when((i == NR - 1) & (j == NV - 1))
    def _fin():
        out_ref[0] = K_CONST - acc_ref[0] / cnt_ref[0]


def kernel(prediction, target):
    pred = prediction.reshape(N_ROWS, V)
    tgt = target.reshape(NR, 1, R_BLK).astype(jnp.int32)
    out = pl.pallas_call(
        _loss_body,
        grid=(NR, NV),
        in_specs=[
            pl.BlockSpec((1, 1, R_BLK), lambda i, j: (i, 0, 0)),
            pl.BlockSpec((R_BLK, V_BLK), lambda i, j: (i, j)),
        ],
        out_specs=pl.BlockSpec(memory_space=pltpu.SMEM),
        out_shape=jax.ShapeDtypeStruct((1,), jnp.float32),
        scratch_shapes=[
            pltpu.SMEM((1,), jnp.float32),
            pltpu.SMEM((1,), jnp.float32),
        ],
    )(tgt, pred)
    return out[0]


# speedup vs baseline: 1.0672x; 1.0012x over previous
import math

import jax
import jax.numpy as jnp
from jax.experimental import pallas as pl
from jax.experimental.pallas import tpu as pltpu

V = 32000
SMOOTH_A = 0.1 / (V - 1)
CONF_C = 0.9
K_CONST = (V - 1) * SMOOTH_A * math.log(SMOOTH_A) + CONF_C * math.log(CONF_C)

R_BLK = 256
V_BLK = 32000
N_ROWS = 4096
NR = N_ROWS // R_BLK
NV = V // V_BLK


def _loss_body(tgt_ref, pred_ref, out_ref, acc_ref, cnt_ref):
    i = pl.program_id(0)
    j = pl.program_id(1)

    @pl.when((i == 0) & (j == 0))
    def _init():
        acc_ref[0] = 0.0
        cnt_ref[0] = 0.0

    tgt = tgt_ref[0, 0, :]
    maskf = (tgt > 0).astype(jnp.float32)

    @pl.when(j == 0)
    def _count():
        cnt_ref[0] += jnp.sum(maskf)

    pred = pred_ref[...]
    tloc = tgt - j * V_BLK
    col0 = jax.lax.broadcasted_iota(jnp.int32, (R_BLK, V_BLK), 1)
    w = jnp.where(col0 == tloc[:, None], CONF_C, SMOOTH_A)
    row_part = jnp.sum(pred * w, axis=1)
    acc_ref[0] += jnp.sum(row_part * maskf)

    @pl.when((i == NR - 1) & (j == NV - 1))
    def _fin():
        out_ref[0] = K_CONST - acc_ref[0] / cnt_ref[0]


def kernel(prediction, target):
    pred = prediction.reshape(N_ROWS, V)
    tgt = target.reshape(NR, 1, R_BLK).astype(jnp.int32)
    out = pl.pallas_call(
        _loss_body,
        grid=(NR, NV),
        compiler_params=pltpu.CompilerParams(
            vmem_limit_bytes=100 * 1024 * 1024),
        in_specs=[
            pl.BlockSpec((1, 1, R_BLK), lambda i, j: (i, 0, 0)),
            pl.BlockSpec((R_BLK, V_BLK), lambda i, j: (i, j)),
        ],
        out_specs=pl.BlockSpec(memory_space=pltpu.SMEM),
        out_shape=jax.ShapeDtypeStruct((1,), jnp.float32),
        scratch_shapes=[
            pltpu.SMEM((1,), jnp.float32),
            pltpu.SMEM((1,), jnp.float32),
        ],
    )(tgt, pred)
    return out[0]


# speedup vs baseline: 1.0715x; 1.0040x over previous
import math

import jax
import jax.numpy as jnp
from jax.experimental import pallas as pl
from jax.experimental.pallas import tpu as pltpu

V = 32000
SMOOTH_A = 0.1 / (V - 1)
CONF_C = 0.9
K_CONST = (V - 1) * SMOOTH_A * math.log(SMOOTH_A) + CONF_C * math.log(CONF_C)

R_BLK = 512
V_BLK = 6400
N_ROWS = 4096
HALF = N_ROWS // 2
NR = HALF // R_BLK
NV = V // V_BLK


def _loss_body(tgt_ref, pa_ref, pb_ref, out_ref, acc_ref, cnt_ref):
    i = pl.program_id(0)
    j = pl.program_id(1)

    @pl.when((i == 0) & (j == 0))
    def _init():
        acc_ref[0] = 0.0
        cnt_ref[0] = 0.0

    tgt = tgt_ref[0, 0, :]

    @pl.when(j == 0)
    def _count():
        cnt_ref[0] += jnp.sum((tgt > 0).astype(jnp.float32))

    col0 = jax.lax.broadcasted_iota(jnp.int32, (R_BLK, V_BLK), 1)
    s = 0.0
    for k, ref in ((0, pa_ref), (1, pb_ref)):
        tg = tgt[k * R_BLK:(k + 1) * R_BLK]
        maskf = (tg > 0).astype(jnp.float32)
        tloc = tg - j * V_BLK
        w = jnp.where(col0 == tloc[:, None], CONF_C, SMOOTH_A)
        row_part = jnp.sum(ref[...] * w, axis=1)
        s = s + jnp.sum(row_part * maskf)
    acc_ref[0] += s

    @pl.when((i == NR - 1) & (j == NV - 1))
    def _fin():
        out_ref[0] = K_CONST - acc_ref[0] / cnt_ref[0]


def kernel(prediction, target):
    pred = prediction.reshape(N_ROWS, V)
    tgt = target.reshape(N_ROWS).astype(jnp.int32)
    tgt2 = jnp.concatenate(
        [tgt[:HALF].reshape(NR, 1, R_BLK), tgt[HALF:].reshape(NR, 1, R_BLK)],
        axis=2)
    out = pl.pallas_call(
        _loss_body,
        grid=(NR, NV),
        compiler_params=pltpu.CompilerParams(
            vmem_limit_bytes=100 * 1024 * 1024),
        in_specs=[
            pl.BlockSpec((1, 1, 2 * R_BLK), lambda i, j: (i, 0, 0)),
            pl.BlockSpec((R_BLK, V_BLK), lambda i, j: (i, j)),
            pl.BlockSpec((R_BLK, V_BLK), lambda i, j: (i + NR, j)),
        ],
        out_specs=pl.BlockSpec(memory_space=pltpu.SMEM),
        out_shape=jax.ShapeDtypeStruct((1,), jnp.float32),
        scratch_shapes=[
            pltpu.SMEM((1,), jnp.float32),
            pltpu.SMEM((1,), jnp.float32),
        ],
    )(tgt2, pred, pred)
    return out[0]
